# baseline (speedup 1.0000x reference)
import functools

import jax
import jax.numpy as jnp
from jax import lax
from jax.experimental import pallas as pl
from jax.experimental.pallas import tpu as pltpu
from jax.experimental.pallas import tpu_sc as plsc

B, L, W = 1024, 50, 16
D_EMB = 64
CHAR_VOCAB = 100
D_OUT = 2 * D_EMB
N = B * L

_NCHUNKS = 2
_NB = N // _NCHUNKS
_NC, _NS = 2, 16
_NW = _NC * _NS
_ROWS_PER_W = _NB // _NW
_CHUNK = 100
_NIDX = _ROWS_PER_W // _CHUNK


def _glove_body(table_hbm, idx_hbm, out_hbm, idx_v, rows_v, sem):
    wid = lax.axis_index("s") * _NC + lax.axis_index("c")
    base = wid * _ROWS_PER_W
    pltpu.sync_copy(idx_hbm.at[pl.ds(wid * _NIDX, _NIDX)], idx_v)
    copies = [
        pltpu.async_copy(
            table_hbm.at[idx_v.at[j]],
            rows_v.at[pl.ds(j * _CHUNK, _CHUNK)],
            sem,
        )
        for j in range(_NIDX)
    ]
    for c in copies:
        c.wait()
    pltpu.sync_copy(rows_v, out_hbm.at[pl.ds(base, _ROWS_PER_W)])


def _glove_gather(table, idx2d):
    mesh = plsc.VectorSubcoreMesh(core_axis_name="c", subcore_axis_name="s")
    return pl.kernel(
        _glove_body,
        out_type=jax.ShapeDtypeStruct((_NB, D_EMB), jnp.float32),
        mesh=mesh,
        scratch_types=[
            pltpu.VMEM((_NIDX, _CHUNK), jnp.int32),
            pltpu.VMEM((_ROWS_PER_W, D_EMB), jnp.float32),
            pltpu.SemaphoreType.DMA,
        ],
        compiler_params=pltpu.CompilerParams(use_tc_tiling_on_sc=False),
    )(table, idx2d)


_BB = 32
_M = _BB * L
_CVP = 104
_BCHUNK = B // _NCHUNKS
_STEPS = _BCHUNK // _BB


def _mlp_body(cx_ref, ge_ref, sel_ref, im_ref, tbl2_ref, w1_ref, b1_ref,
              w2_ref, b2_ref, out_ref):
    cx2 = cx_ref[...].reshape(_M, W).astype(jnp.bfloat16)
    dr = jnp.dot(cx2, sel_ref[...], preferred_element_type=jnp.float32)
    oh = (dr == im_ref[...]).astype(jnp.bfloat16)
    tbl2 = tbl2_ref[...]
    ce2 = jnp.full((_M, 2 * D_EMB), -jnp.inf, jnp.float32)
    for p in range(W // 2):
        ce2 = jnp.maximum(ce2, jnp.dot(oh[:, p * 2 * _CVP:(p + 1) * 2 * _CVP],
                                       tbl2, preferred_element_type=jnp.float32))
    ce = jnp.maximum(ce2[:, :D_EMB], ce2[:, D_EMB:])
    h = jnp.concatenate([ce, ge_ref[...]], axis=-1)

    def highway(hh, wc, bc):
        og = jnp.dot(hh, wc, preferred_element_type=jnp.float32) + bc
        o = jnp.maximum(og[:, :D_OUT], 0.0)
        g = 1.0 / (1.0 + jnp.exp(-og[:, D_OUT:]))
        return hh * g + o * (1.0 - g)

    h = highway(h, w1_ref[...], b1_ref[...])
    h = highway(h, w2_ref[...], b2_ref[...])
    out_ref[...] = h.reshape(_BB, L, D_OUT)


def _mlp_chunk(c, cx, ge, prev, sel, im, tbl2, w1, b1, w2, b2):
    full = lambda shape: pl.BlockSpec(shape, lambda i: (0, 0))
    args = [cx, ge, sel, im, tbl2, w1, b1, w2, b2]
    in_specs = [
        pl.BlockSpec((_BB, L, W), lambda i: (c * _STEPS + i, 0, 0)),
        pl.BlockSpec((_M, D_EMB), lambda i: (i, 0)),
        full((W, W * _CVP)),
        full((1, W * _CVP)),
        full((2 * _CVP, 2 * D_EMB)),
        full((D_OUT, 2 * D_OUT)), full((1, 2 * D_OUT)),
        full((D_OUT, 2 * D_OUT)), full((1, 2 * D_OUT)),
    ]
    body = _mlp_body
    aliases = {}
    if prev is not None:
        args = [prev] + args
        in_specs = [pl.BlockSpec(memory_space=pltpu.MemorySpace.HBM)] + in_specs
        aliases = {0: 0}
        body = lambda _prev_ref, *refs: _mlp_body(*refs)
    return pl.pallas_call(
        body,
        grid=(_STEPS,),
        in_specs=in_specs,
        out_specs=pl.BlockSpec((_BB, L, D_OUT), lambda i: (c * _STEPS + i, 0, 0)),
        out_shape=jax.ShapeDtypeStruct((B, L, D_OUT), jnp.float32),
        input_output_aliases=aliases,
        compiler_params=pltpu.CompilerParams(
            dimension_semantics=("arbitrary",),
            vmem_limit_bytes=100 * 1024 * 1024,
        ),
    )(*args)


def _dense_consts(char_table, W_i1, b_i1, W_g1, b_g1, W_i2, b_i2, W_g2, b_g2):
    sel = jnp.repeat(jnp.eye(W, dtype=jnp.bfloat16), _CVP, axis=1)
    im = (jnp.arange(W * _CVP) % _CVP).astype(jnp.float32).reshape(1, -1)
    tblp = jnp.zeros((_CVP, D_EMB), jnp.bfloat16).at[:CHAR_VOCAB].set(
        char_table.astype(jnp.bfloat16))
    z = jnp.zeros_like(tblp)
    tbl2 = jnp.block([[tblp, z], [z, tblp]])
    w1 = jnp.concatenate([W_i1.T, W_g1.T], axis=1)
    w2 = jnp.concatenate([W_i2.T, W_g2.T], axis=1)
    b1 = jnp.concatenate([b_i1, b_g1]).reshape(1, 2 * D_OUT)
    b2 = jnp.concatenate([b_i2, b_g2]).reshape(1, 2 * D_OUT)
    return sel, im, tbl2, w1, b1, w2, b2


def kernel(cx, gx, x, char_table, glove_table, W_i1, b_i1, W_g1, b_g1,
           W_i2, b_i2, W_g2, b_g2):
    del x
    idx = gx.astype(jnp.int32).reshape(_NCHUNKS, _NB // _CHUNK, _CHUNK)
    consts = _dense_consts(
        char_table, W_i1, b_i1, W_g1, b_g1, W_i2, b_i2, W_g2, b_g2)
    ges = [_glove_gather(glove_table, idx[c]) for c in range(_NCHUNKS)]
    out = None
    for c in range(_NCHUNKS):
        out = _mlp_chunk(c, cx, ges[c], out, *consts)
    return out


# speedup vs baseline: 1.0262x; 1.0262x over previous
import functools

import jax
import jax.numpy as jnp
from jax import lax
from jax.experimental import pallas as pl
from jax.experimental.pallas import tpu as pltpu
from jax.experimental.pallas import tpu_sc as plsc

B, L, W = 1024, 50, 16
D_EMB = 64
CHAR_VOCAB = 100
D_OUT = 2 * D_EMB
N = B * L

_NCHUNKS = 1
_NB = N // _NCHUNKS
_NC, _NS = 2, 16
_NW = _NC * _NS
_ROWS_PER_W = _NB // _NW
_CHUNK = 100
_NIDX = _ROWS_PER_W // _CHUNK


def _glove_body(table_hbm, idx_hbm, out_hbm, idx_v, rows_v, sem):
    wid = lax.axis_index("s") * _NC + lax.axis_index("c")
    base = wid * _ROWS_PER_W
    pltpu.sync_copy(idx_hbm.at[pl.ds(wid * _NIDX, _NIDX)], idx_v)
    copies = [
        pltpu.async_copy(
            table_hbm.at[idx_v.at[j]],
            rows_v.at[pl.ds(j * _CHUNK, _CHUNK)],
            sem,
        )
        for j in range(_NIDX)
    ]
    for c in copies:
        c.wait()
    pltpu.sync_copy(rows_v, out_hbm.at[pl.ds(base, _ROWS_PER_W)])


def _glove_gather(table, idx2d):
    mesh = plsc.VectorSubcoreMesh(core_axis_name="c", subcore_axis_name="s")
    return pl.kernel(
        _glove_body,
        out_type=jax.ShapeDtypeStruct((_NB, D_EMB), jnp.float32),
        mesh=mesh,
        scratch_types=[
            pltpu.VMEM((_NIDX, _CHUNK), jnp.int32),
            pltpu.VMEM((_ROWS_PER_W, D_EMB), jnp.float32),
            pltpu.SemaphoreType.DMA,
        ],
        compiler_params=pltpu.CompilerParams(use_tc_tiling_on_sc=False),
    )(table, idx2d)


_BB = 32
_M = _BB * L
_CVP = 104
_BCHUNK = B // _NCHUNKS
_STEPS = _BCHUNK // _BB


def _mlp_body(cx_ref, ge_ref, sel_ref, im_ref, tbl2_ref, w1_ref, b1_ref,
              w2_ref, b2_ref, out_ref):
    cx2 = cx_ref[...].reshape(_M, W).astype(jnp.bfloat16)
    dr = jnp.dot(cx2, sel_ref[...], preferred_element_type=jnp.float32)
    oh = (dr == im_ref[...]).astype(jnp.bfloat16)
    tbl2 = tbl2_ref[...]
    ce2 = jnp.full((_M, 2 * D_EMB), -jnp.inf, jnp.float32)
    for p in range(W // 2):
        ce2 = jnp.maximum(ce2, jnp.dot(oh[:, p * 2 * _CVP:(p + 1) * 2 * _CVP],
                                       tbl2, preferred_element_type=jnp.float32))
    ce = jnp.maximum(ce2[:, :D_EMB], ce2[:, D_EMB:])
    h = jnp.concatenate([ce, ge_ref[...]], axis=-1)

    def highway(hh, wc, bc):
        og = jnp.dot(hh, wc, preferred_element_type=jnp.float32) + bc
        o = jnp.maximum(og[:, :D_OUT], 0.0)
        g = 1.0 / (1.0 + jnp.exp(-og[:, D_OUT:]))
        return hh * g + o * (1.0 - g)

    h = highway(h, w1_ref[...], b1_ref[...])
    h = highway(h, w2_ref[...], b2_ref[...])
    out_ref[...] = h.reshape(_BB, L, D_OUT)


def _mlp_chunk(c, cx, ge, prev, sel, im, tbl2, w1, b1, w2, b2):
    full = lambda shape: pl.BlockSpec(shape, lambda i: (0, 0))
    args = [cx, ge, sel, im, tbl2, w1, b1, w2, b2]
    in_specs = [
        pl.BlockSpec((_BB, L, W), lambda i: (c * _STEPS + i, 0, 0)),
        pl.BlockSpec((_M, D_EMB), lambda i: (i, 0)),
        full((W, W * _CVP)),
        full((1, W * _CVP)),
        full((2 * _CVP, 2 * D_EMB)),
        full((D_OUT, 2 * D_OUT)), full((1, 2 * D_OUT)),
        full((D_OUT, 2 * D_OUT)), full((1, 2 * D_OUT)),
    ]
    body = _mlp_body
    aliases = {}
    if prev is not None:
        args = [prev] + args
        in_specs = [pl.BlockSpec(memory_space=pltpu.MemorySpace.HBM)] + in_specs
        aliases = {0: 0}
        body = lambda _prev_ref, *refs: _mlp_body(*refs)
    return pl.pallas_call(
        body,
        grid=(_STEPS,),
        in_specs=in_specs,
        out_specs=pl.BlockSpec((_BB, L, D_OUT), lambda i: (c * _STEPS + i, 0, 0)),
        out_shape=jax.ShapeDtypeStruct((B, L, D_OUT), jnp.float32),
        input_output_aliases=aliases,
        compiler_params=pltpu.CompilerParams(
            dimension_semantics=("arbitrary",),
            vmem_limit_bytes=100 * 1024 * 1024,
        ),
    )(*args)


def _dense_consts(char_table, W_i1, b_i1, W_g1, b_g1, W_i2, b_i2, W_g2, b_g2):
    sel = jnp.repeat(jnp.eye(W, dtype=jnp.bfloat16), _CVP, axis=1)
    im = (jnp.arange(W * _CVP) % _CVP).astype(jnp.float32).reshape(1, -1)
    tblp = jnp.zeros((_CVP, D_EMB), jnp.bfloat16).at[:CHAR_VOCAB].set(
        char_table.astype(jnp.bfloat16))
    z = jnp.zeros_like(tblp)
    tbl2 = jnp.block([[tblp, z], [z, tblp]])
    w1 = jnp.concatenate([W_i1.T, W_g1.T], axis=1)
    w2 = jnp.concatenate([W_i2.T, W_g2.T], axis=1)
    b1 = jnp.concatenate([b_i1, b_g1]).reshape(1, 2 * D_OUT)
    b2 = jnp.concatenate([b_i2, b_g2]).reshape(1, 2 * D_OUT)
    return sel, im, tbl2, w1, b1, w2, b2


def kernel(cx, gx, x, char_table, glove_table, W_i1, b_i1, W_g1, b_g1,
           W_i2, b_i2, W_g2, b_g2):
    del x
    idx = gx.astype(jnp.int32).reshape(_NCHUNKS, _NB // _CHUNK, _CHUNK)
    consts = _dense_consts(
        char_table, W_i1, b_i1, W_g1, b_g1, W_i2, b_i2, W_g2, b_g2)
    ges = [_glove_gather(glove_table, idx[c]) for c in range(_NCHUNKS)]
    out = None
    for c in range(_NCHUNKS):
        out = _mlp_chunk(c, cx, ges[c], out, *consts)
    return out
